# baseline (speedup 1.0000x reference)
import functools

import jax
import jax.numpy as jnp
from jax import lax
from jax.experimental import pallas as pl
from jax.experimental.pallas import tpu as pltpu
from jax.experimental.pallas import tpu_sc as plsc

D = 64
BBLK = 128
NJ = 200
NBT = 32
SCALE = 8.0


def _sc_gather_scaled(table, x_view):
    mesh = plsc.VectorSubcoreMesh(core_axis_name="core", subcore_axis_name="subcore")

    @functools.partial(
        pl.kernel,
        out_type=jax.ShapeDtypeStruct((NJ, 8, NBT, 8, BBLK), jnp.float32),
        mesh=mesh,
        scratch_types=[
            pltpu.VMEM((25, 8, BBLK), jnp.int32),
            pltpu.VMEM((BBLK, D), jnp.float32),
            pltpu.VMEM((BBLK, D), jnp.float32),
            pltpu.VMEM((8, 8, BBLK), jnp.float32),
            pltpu.VMEM((8, 8, BBLK), jnp.float32),
            pltpu.SemaphoreType.DMA,
            pltpu.SemaphoreType.DMA,
            pltpu.SemaphoreType.DMA,
            pltpu.SemaphoreType.DMA,
        ],
        compiler_params=pltpu.CompilerParams(
            use_tc_tiling_on_sc=False, needs_layout_passes=False
        ),
    )
    def k(table_hbm, x_hbm, out_hbm, idxv, rows0, rows1, ob0, ob1, sg0, sg1, so0, so1):
        w = lax.axis_index("subcore") * 2 + lax.axis_index("core")
        rows = (rows0, rows1)
        obuf = (ob0, ob1)
        sg = (sg0, sg1)
        so = (so0, so1)

        pltpu.sync_copy(x_hbm.at[:, w], idxv)

        iota = lax.iota(jnp.int32, 16)
        rowsel = [iota + (16 * g) for g in range(8)]

        def gstart(j, b):
            pltpu.make_async_copy(
                table_hbm.at[idxv.at[j // 8, j % 8]], rows[b], sg[b]
            ).start()

        def gwait(b):
            pltpu.make_async_copy(
                table_hbm.at[idxv.at[0, 0]], rows[b], sg[b]
            ).wait()

        def ostart(j, b):
            pltpu.make_async_copy(obuf[b], out_hbm.at[j, :, w], so[b]).start()

        def owait(b):
            pltpu.make_async_copy(obuf[b], out_hbm.at[0, :, w], so[b]).wait()

        def transpose_scale(b):
            @pl.loop(0, D)
            def _(c):
                colv = jnp.full((16,), 0, jnp.int32) + c
                ct = c // 8
                cin = c % 8
                for g in range(8):
                    v = plsc.load_gather(rows[b], [rowsel[g], colv])
                    obuf[b][ct, cin, pl.ds(16 * g, 16)] = v * SCALE

        gstart(0, 0)

        @pl.loop(0, NJ, step=2)
        def _(jj):
            for b in range(2):
                j = jj + b

                @pl.when(j + 1 < NJ)
                def _():
                    gstart(j + 1, 1 - b)

                gwait(b)

                @pl.when(j >= 2)
                def _():
                    owait(b)

                transpose_scale(b)
                ostart(j, b)

        owait(0)
        owait(1)

    return k(table, x_view)


def kernel(x, table):
    x_view = x.astype(jnp.int32).reshape(32, 128, 25, 8).transpose(2, 0, 3, 1)
    out5 = _sc_gather_scaled(table, x_view)
    return out5.transpose(2, 4, 0, 1, 3).reshape(4096, 200, 64)
